# Initial kernel scaffold; baseline (speedup 1.0000x reference)
import jax
import jax.numpy as jnp
from jax.experimental import pallas as pl


def kernel(p1, x1, p2, x2, W1, g1, b1, W2, g2, b2):
    raise NotImplementedError("write your pallas kernel here")


# speedup vs baseline: 23.9684x; 23.9684x over previous
import functools

import jax
import jax.numpy as jnp
from jax import lax
from jax.experimental import pallas as pl
from jax.experimental.pallas import tpu as pltpu

_EPSILON = 1e-8
_GN_EPS = 1e-5
_BQ = 512
_HI = lax.Precision.HIGHEST


def _group_scale_shift(h, gamma, beta, group_size):
    C = h.shape[0]
    n = group_size * h.shape[1]
    s = jnp.sum(h, axis=1, keepdims=True)
    q = jnp.sum(h * h, axis=1, keepdims=True)
    gi = lax.broadcasted_iota(jnp.int32, (C, C), 0) // group_size
    gj = lax.broadcasted_iota(jnp.int32, (C, C), 1) // group_size
    G = (gi == gj).astype(jnp.float32)
    gs = lax.dot_general(G, s, (((1,), (0,)), ((), ())), precision=_HI,
                         preferred_element_type=jnp.float32)
    gq = lax.dot_general(G, q, (((1,), (0,)), ((), ())), precision=_HI,
                         preferred_element_type=jnp.float32)
    mean = gs / n
    var = gq / n - mean * mean
    rstd = lax.rsqrt(var + _GN_EPS)
    scale = gamma * rstd
    shift = beta - mean * scale
    return scale, shift


def _kA(x1_ref, x2_ref, w1_ref, g1_ref, b1_ref, w2_ref, g2_ref, b2_ref,
        x1l_ref, sc2_ref, sh2_ref, *, group_size):
    f32 = jnp.float32
    h1 = lax.dot_general(w1_ref[...], x1_ref[...], (((1,), (0,)), ((), ())),
                         preferred_element_type=f32)
    sc1, sh1 = _group_scale_shift(h1, g1_ref[...], b1_ref[...], group_size)
    x1l_ref[...] = jnp.maximum(h1 * sc1 + sh1, 0.0)

    h2 = lax.dot_general(w2_ref[...], x2_ref[...], (((1,), (0,)), ((), ())),
                         preferred_element_type=f32)
    sc2, sh2 = _group_scale_shift(h2, g2_ref[...], b2_ref[...], group_size)
    sc2_ref[...] = sc2
    sh2_ref[...] = sh2


def _kB(p1_ref, p1t_ref, p2_ref, x1l_ref, x2_ref, w2_ref, sc2_ref, sh2_ref,
        y_ref):
    f32 = jnp.float32
    p1 = p1_ref[...]
    p1t = p1t_ref[...]
    p2 = p2_ref[...]
    n1c = p1.shape[0]
    cross = lax.dot_general(p2, p1, (((1,), (1,)), ((), ())),
                            preferred_element_type=f32)
    n2col = jnp.sum(p2 * p2, axis=1, keepdims=True)
    n1row = jnp.sum(p1t * p1t, axis=0, keepdims=True)
    d2 = jnp.maximum((n2col + n1row) - 2.0 * cross, 0.0)

    iota = lax.broadcasted_iota(jnp.int32, d2.shape, 1)
    ms, idxs = [], []
    for k in range(3):
        m = jnp.min(d2, axis=1, keepdims=True)
        idx = jnp.min(jnp.where(d2 == m, iota, n1c), axis=1,
                      keepdims=True)
        ms.append(m)
        idxs.append(idx)
        if k < 2:
            d2 = jnp.where(iota == idx, f32(1e30), d2)

    r = [1.0 / (m + _EPSILON) for m in ms]
    norm = r[0] + r[1] + r[2]
    w = [ri / norm for ri in r]
    zero = jnp.zeros_like(d2)
    wmat = (jnp.where(iota == idxs[0], w[0], zero)
            + jnp.where(iota == idxs[1], w[1], zero)
            + jnp.where(iota == idxs[2], w[2], zero))

    up = lax.dot_general(x1l_ref[...], wmat, (((1,), (1,)), ((), ())),
                         precision=_HI, preferred_element_type=f32)
    h2 = lax.dot_general(w2_ref[...], x2_ref[...], (((1,), (0,)), ((), ())),
                         preferred_element_type=f32)
    x2l = jnp.maximum(h2 * sc2_ref[...] + sh2_ref[...], 0.0)
    y_ref[...] = x2l + up


def kernel(p1, x1, p2, x2, W1, g1, b1, W2, g2, b2):
    B, N1, _ = p1.shape
    N2 = p2.shape[1]
    Cin = x1.shape[1]
    Cs = x2.shape[1]
    C = W1.shape[0]
    group_size = 16
    bq = _BQ

    p1t = jnp.transpose(p1, (0, 2, 1))
    g1c = g1.reshape(C, 1)
    b1c = b1.reshape(C, 1)
    g2c = g2.reshape(C, 1)
    b2c = b2.reshape(C, 1)

    x1l, sc2, sh2 = pl.pallas_call(
        functools.partial(_kA, group_size=group_size),
        grid=(B,),
        in_specs=[
            pl.BlockSpec((None, Cin, N1), lambda b: (b, 0, 0)),
            pl.BlockSpec((None, Cs, N2), lambda b: (b, 0, 0)),
            pl.BlockSpec((C, Cin), lambda b: (0, 0)),
            pl.BlockSpec((C, 1), lambda b: (0, 0)),
            pl.BlockSpec((C, 1), lambda b: (0, 0)),
            pl.BlockSpec((C, Cs), lambda b: (0, 0)),
            pl.BlockSpec((C, 1), lambda b: (0, 0)),
            pl.BlockSpec((C, 1), lambda b: (0, 0)),
        ],
        out_specs=[
            pl.BlockSpec((None, C, N1), lambda b: (b, 0, 0)),
            pl.BlockSpec((None, C, 1), lambda b: (b, 0, 0)),
            pl.BlockSpec((None, C, 1), lambda b: (b, 0, 0)),
        ],
        out_shape=[
            jax.ShapeDtypeStruct((B, C, N1), jnp.float32),
            jax.ShapeDtypeStruct((B, C, 1), jnp.float32),
            jax.ShapeDtypeStruct((B, C, 1), jnp.float32),
        ],
    )(x1, x2, W1, g1c, b1c, W2, g2c, b2c)

    y = pl.pallas_call(
        _kB,
        grid=(B, N2 // bq),
        in_specs=[
            pl.BlockSpec((None, N1, 3), lambda b, q: (b, 0, 0)),
            pl.BlockSpec((None, 3, N1), lambda b, q: (b, 0, 0)),
            pl.BlockSpec((None, bq, 3), lambda b, q: (b, q, 0)),
            pl.BlockSpec((None, C, N1), lambda b, q: (b, 0, 0)),
            pl.BlockSpec((None, Cs, bq), lambda b, q: (b, 0, q)),
            pl.BlockSpec((C, Cs), lambda b, q: (0, 0)),
            pl.BlockSpec((None, C, 1), lambda b, q: (b, 0, 0)),
            pl.BlockSpec((None, C, 1), lambda b, q: (b, 0, 0)),
        ],
        out_specs=pl.BlockSpec((None, C, bq), lambda b, q: (b, 0, q)),
        out_shape=jax.ShapeDtypeStruct((B, C, N2), jnp.float32),
    )(p1, p1t, p2, x1l, x2, W2, sc2, sh2)

    return (p2, y)


# speedup vs baseline: 53.5140x; 2.2327x over previous
import functools

import jax
import jax.numpy as jnp
from jax import lax
from jax.experimental import pallas as pl
from jax.experimental.pallas import tpu as pltpu

_EPSILON = 1e-8
_GN_EPS = 1e-5
_BQ = 1024
_HI = lax.Precision.HIGHEST


def _group_scale_shift(h, gamma, beta, group_size):
    C = h.shape[0]
    n = group_size * h.shape[1]
    s = jnp.sum(h, axis=1, keepdims=True)
    q = jnp.sum(h * h, axis=1, keepdims=True)
    gi = lax.broadcasted_iota(jnp.int32, (C, C), 0) // group_size
    gj = lax.broadcasted_iota(jnp.int32, (C, C), 1) // group_size
    G = (gi == gj).astype(jnp.float32)
    gs = lax.dot_general(G, s, (((1,), (0,)), ((), ())), precision=_HI,
                         preferred_element_type=jnp.float32)
    gq = lax.dot_general(G, q, (((1,), (0,)), ((), ())), precision=_HI,
                         preferred_element_type=jnp.float32)
    mean = gs / n
    var = gq / n - mean * mean
    rstd = lax.rsqrt(var + _GN_EPS)
    scale = gamma * rstd
    shift = beta - mean * scale
    return scale, shift


def _kA(x1_ref, x2_ref, w1_ref, g1_ref, b1_ref, w2_ref, g2_ref, b2_ref,
        x1l_ref, sc2_ref, sh2_ref, *, group_size):
    f32 = jnp.float32
    h1 = lax.dot_general(w1_ref[...], x1_ref[...], (((1,), (0,)), ((), ())),
                         preferred_element_type=f32)
    sc1, sh1 = _group_scale_shift(h1, g1_ref[...], b1_ref[...], group_size)
    x1l_ref[...] = jnp.maximum(h1 * sc1 + sh1, 0.0)

    h2 = lax.dot_general(w2_ref[...], x2_ref[...], (((1,), (0,)), ((), ())),
                         preferred_element_type=f32)
    sc2, sh2 = _group_scale_shift(h2, g2_ref[...], b2_ref[...], group_size)
    sc2_ref[...] = sc2
    sh2_ref[...] = sh2


def _kB(p1_ref, p1t_ref, p2_ref, x1l_ref, x2_ref, w2_ref, sc2_ref, sh2_ref,
        y_ref):
    f32 = jnp.float32
    p1 = p1_ref[...]
    p1t = p1t_ref[...]
    p2 = p2_ref[...]
    n1c = p1.shape[0]
    cross = lax.dot_general(p2, p1, (((1,), (1,)), ((), ())),
                            preferred_element_type=f32)
    n2col = jnp.sum(p2 * p2, axis=1, keepdims=True)
    n1row = jnp.sum(p1t * p1t, axis=0, keepdims=True)
    d2raw = (n2col + n1row) - 2.0 * cross

    iota_f = lax.broadcasted_iota(jnp.int32, d2raw.shape, 1).astype(f32)
    d2 = jnp.where(d2raw <= 0.0, iota_f * f32(1e-30), d2raw)

    ms = []
    dwork = d2
    for k in range(3):
        m = jnp.min(dwork, axis=1, keepdims=True)
        ms.append(m)
        if k < 2:
            dwork = jnp.where(dwork == m, f32(1e30), dwork)

    r = [1.0 / (m + _EPSILON) for m in ms]
    norm = r[0] + r[1] + r[2]
    w = [ri / norm for ri in r]
    zero = jnp.zeros_like(d2)
    wmat = (jnp.where(d2 == ms[0], w[0], zero)
            + jnp.where(d2 == ms[1], w[1], zero)
            + jnp.where(d2 == ms[2], w[2], zero))

    up = lax.dot_general(x1l_ref[...], wmat, (((1,), (1,)), ((), ())),
                         preferred_element_type=f32)
    h2 = lax.dot_general(w2_ref[...], x2_ref[...], (((1,), (0,)), ((), ())),
                         preferred_element_type=f32)
    x2l = jnp.maximum(h2 * sc2_ref[...] + sh2_ref[...], 0.0)
    y_ref[...] = x2l + up


def kernel(p1, x1, p2, x2, W1, g1, b1, W2, g2, b2):
    B, N1, _ = p1.shape
    N2 = p2.shape[1]
    Cin = x1.shape[1]
    Cs = x2.shape[1]
    C = W1.shape[0]
    group_size = 16
    bq = _BQ

    p1t = jnp.transpose(p1, (0, 2, 1))
    g1c = g1.reshape(C, 1)
    b1c = b1.reshape(C, 1)
    g2c = g2.reshape(C, 1)
    b2c = b2.reshape(C, 1)

    x1l, sc2, sh2 = pl.pallas_call(
        functools.partial(_kA, group_size=group_size),
        grid=(B,),
        in_specs=[
            pl.BlockSpec((None, Cin, N1), lambda b: (b, 0, 0)),
            pl.BlockSpec((None, Cs, N2), lambda b: (b, 0, 0)),
            pl.BlockSpec((C, Cin), lambda b: (0, 0)),
            pl.BlockSpec((C, 1), lambda b: (0, 0)),
            pl.BlockSpec((C, 1), lambda b: (0, 0)),
            pl.BlockSpec((C, Cs), lambda b: (0, 0)),
            pl.BlockSpec((C, 1), lambda b: (0, 0)),
            pl.BlockSpec((C, 1), lambda b: (0, 0)),
        ],
        out_specs=[
            pl.BlockSpec((None, C, N1), lambda b: (b, 0, 0)),
            pl.BlockSpec((None, C, 1), lambda b: (b, 0, 0)),
            pl.BlockSpec((None, C, 1), lambda b: (b, 0, 0)),
        ],
        out_shape=[
            jax.ShapeDtypeStruct((B, C, N1), jnp.float32),
            jax.ShapeDtypeStruct((B, C, 1), jnp.float32),
            jax.ShapeDtypeStruct((B, C, 1), jnp.float32),
        ],
    )(x1, x2, W1, g1c, b1c, W2, g2c, b2c)

    y = pl.pallas_call(
        _kB,
        grid=(B, N2 // bq),
        in_specs=[
            pl.BlockSpec((None, N1, 3), lambda b, q: (b, 0, 0)),
            pl.BlockSpec((None, 3, N1), lambda b, q: (b, 0, 0)),
            pl.BlockSpec((None, bq, 3), lambda b, q: (b, q, 0)),
            pl.BlockSpec((None, C, N1), lambda b, q: (b, 0, 0)),
            pl.BlockSpec((None, Cs, bq), lambda b, q: (b, 0, q)),
            pl.BlockSpec((C, Cs), lambda b, q: (0, 0)),
            pl.BlockSpec((None, C, 1), lambda b, q: (b, 0, 0)),
            pl.BlockSpec((None, C, 1), lambda b, q: (b, 0, 0)),
        ],
        out_specs=pl.BlockSpec((None, C, bq), lambda b, q: (b, 0, q)),
        out_shape=jax.ShapeDtypeStruct((B, C, N2), jnp.float32),
    )(p1, p1t, p2, x1l, x2, W2, sc2, sh2)

    return (p2, y)
